# Initial kernel scaffold; baseline (speedup 1.0000x reference)
import jax
import jax.numpy as jnp
from jax.experimental import pallas as pl


def kernel(x, W):
    raise NotImplementedError("write your pallas kernel here")


# speedup vs baseline: 2.0617x; 2.0617x over previous
import functools

import jax
import jax.numpy as jnp
from jax.experimental import pallas as pl

E = 64
N_GROUP = 8
TOPK_GROUP = 3
TOP_K = 8
GROUP_SIZE = E // N_GROUP


def _gate_block(x_ref, w_ref, idx_ref, wgt_ref):
    x = x_ref[...]
    w = w_ref[...]
    logits = jax.lax.dot_general(
        x, w, (((1,), (1,)), ((), ())),
        preferred_element_type=jnp.float32)
    tb = logits.shape[0]

    m = jnp.max(logits, axis=-1, keepdims=True)
    ex = jnp.exp(logits - m)
    scores = ex / jnp.sum(ex, axis=-1, keepdims=True)

    gparts = [
        jnp.max(scores[:, g * GROUP_SIZE:(g + 1) * GROUP_SIZE],
                axis=-1, keepdims=True)
        for g in range(N_GROUP)
    ]
    gmax = jnp.concatenate(gparts, axis=1)

    iota_g = jax.lax.broadcasted_iota(jnp.int32, (tb, N_GROUP), 1)
    work = gmax
    sel = jnp.zeros((tb, N_GROUP), dtype=jnp.float32)
    for _ in range(TOPK_GROUP):
        gm = jnp.max(work, axis=-1, keepdims=True)
        fidx = jnp.min(jnp.where(work == gm, iota_g, N_GROUP),
                       axis=-1, keepdims=True)
        pick = iota_g == fidx
        sel = sel + jnp.where(pick, 1.0, 0.0)
        work = jnp.where(pick, -1.0, work)

    mask64 = jnp.concatenate(
        [jnp.broadcast_to(sel[:, g:g + 1], (tb, GROUP_SIZE))
         for g in range(N_GROUP)], axis=1)
    cand = scores * mask64

    iota_e = jax.lax.broadcasted_iota(jnp.int32, (tb, E), 1)
    idx_cols = []
    wgt_cols = []
    work = cand
    for _ in range(TOP_K):
        km = jnp.max(work, axis=-1, keepdims=True)
        fidx = jnp.min(jnp.where(work == km, iota_e, E),
                       axis=-1, keepdims=True)
        idx_cols.append(fidx)
        wgt_cols.append(km)
        work = jnp.where(iota_e == fidx, -1.0, work)

    idx_ref[...] = jnp.concatenate(idx_cols, axis=1)
    wgt_ref[...] = jnp.concatenate(wgt_cols, axis=1)


@functools.partial(jax.jit, static_argnames=())
def kernel(x, W):
    b, s, h = x.shape
    t = b * s
    xs = x.reshape(t, h)
    tb = 1024
    grid = (t // tb,)
    idx, wgt = pl.pallas_call(
        _gate_block,
        grid=grid,
        in_specs=[
            pl.BlockSpec((tb, h), lambda i: (i, 0)),
            pl.BlockSpec((E, h), lambda i: (0, 0)),
        ],
        out_specs=[
            pl.BlockSpec((tb, TOP_K), lambda i: (i, 0)),
            pl.BlockSpec((tb, TOP_K), lambda i: (i, 0)),
        ],
        out_shape=[
            jax.ShapeDtypeStruct((t, TOP_K), jnp.int32),
            jax.ShapeDtypeStruct((t, TOP_K), jnp.float32),
        ],
    )(xs, W)
    return idx, wgt


# speedup vs baseline: 9.7379x; 4.7232x over previous
import functools

import jax
import jax.numpy as jnp
from jax.experimental import pallas as pl

E = 64
N_GROUP = 8
TOPK_GROUP = 3
TOP_K = 8
GROUP_SIZE = E // N_GROUP


def _gate_block(x_ref, w_ref, idx_ref, wgt_ref):
    x = x_ref[...]
    w = w_ref[...]
    lt = jax.lax.dot_general(
        w, x, (((1,), (1,)), ((), ())),
        preferred_element_type=jnp.float32)
    tb = lt.shape[1]

    m0 = jnp.max(lt, axis=0, keepdims=True)
    ex = jnp.exp(lt - m0)
    den = jnp.sum(ex, axis=0, keepdims=True)
    sc = ex / den

    g = jnp.max(sc.reshape(N_GROUP, GROUP_SIZE, tb), axis=1, keepdims=True)
    gfull = jnp.broadcast_to(
        g, (N_GROUP, GROUP_SIZE, tb)).reshape(E, tb)

    sel = jnp.zeros((E, tb), dtype=jnp.float32)
    work = gfull
    for _ in range(TOPK_GROUP):
        gm = jnp.max(work, axis=0, keepdims=True)
        eq = work == gm
        sel = sel + jnp.where(eq, 1.0, 0.0)
        work = jnp.where(eq, -1.0, work)

    cand = sc * sel

    rowid = jax.lax.broadcasted_iota(jnp.int32, (E, tb), 0)
    work = cand
    for k in range(TOP_K):
        km = jnp.max(work, axis=0, keepdims=True)
        fidx = jnp.min(jnp.where(work == km, rowid, E),
                       axis=0, keepdims=True)
        work = jnp.where(rowid == fidx, -1.0, work)
        idx_ref[k:k + 1, :] = fidx
        wgt_ref[k:k + 1, :] = km


@functools.partial(jax.jit, static_argnames=())
def kernel(x, W):
    b, s, h = x.shape
    t = b * s
    xs = x.reshape(t, h)
    tb = 1024
    grid = (t // tb,)
    idx_t, wgt_t = pl.pallas_call(
        _gate_block,
        grid=grid,
        in_specs=[
            pl.BlockSpec((tb, h), lambda i: (i, 0)),
            pl.BlockSpec((E, h), lambda i: (0, 0)),
        ],
        out_specs=[
            pl.BlockSpec((TOP_K, tb), lambda i: (0, i)),
            pl.BlockSpec((TOP_K, tb), lambda i: (0, i)),
        ],
        out_shape=[
            jax.ShapeDtypeStruct((TOP_K, t), jnp.int32),
            jax.ShapeDtypeStruct((TOP_K, t), jnp.float32),
        ],
    )(xs, W)
    return idx_t.T, wgt_t.T


# speedup vs baseline: 10.1819x; 1.0456x over previous
import functools

import jax
import jax.numpy as jnp
from jax.experimental import pallas as pl

E = 64
N_GROUP = 8
TOPK_GROUP = 3
TOP_K = 8
GROUP_SIZE = E // N_GROUP


def _gate_block(x_ref, w_ref, idx_ref, wgt_ref):
    x = x_ref[...]
    w = w_ref[...]
    lt = jax.lax.dot_general(
        w, x, (((1,), (1,)), ((), ())),
        preferred_element_type=jnp.float32)
    tb = lt.shape[1]
    ninf = jnp.float32(-jnp.inf)

    g = jnp.max(lt.reshape(N_GROUP, GROUP_SIZE, tb), axis=1)
    m0 = jnp.max(g, axis=0, keepdims=True)

    den = jnp.sum(jnp.exp(lt - m0), axis=0, keepdims=True)
    rden = 1.0 / den

    sel = jnp.zeros((N_GROUP, tb), dtype=jnp.float32)
    work = g
    for _ in range(TOPK_GROUP):
        gm = jnp.max(work, axis=0, keepdims=True)
        eq = work == gm
        sel = sel + jnp.where(eq, 1.0, 0.0)
        work = jnp.where(eq, ninf, work)

    sel64 = jnp.broadcast_to(
        sel.reshape(N_GROUP, 1, tb),
        (N_GROUP, GROUP_SIZE, tb)).reshape(E, tb)
    cand = jnp.where(sel64 > 0.0, lt, ninf)

    rowid = jax.lax.broadcasted_iota(jnp.int32, (E, tb), 0)
    work = cand
    for k in range(TOP_K):
        km = jnp.max(work, axis=0, keepdims=True)
        eq = work == km
        fidx = jnp.min(jnp.where(eq, rowid, E),
                       axis=0, keepdims=True)
        work = jnp.where(eq, ninf, work)
        idx_ref[k:k + 1, :] = fidx
        wgt_ref[k:k + 1, :] = jnp.exp(km - m0) * rden


@functools.partial(jax.jit, static_argnames=())
def kernel(x, W):
    b, s, h = x.shape
    t = b * s
    xs = x.reshape(t, h)
    tb = 1024
    grid = (t // tb,)
    idx_t, wgt_t = pl.pallas_call(
        _gate_block,
        grid=grid,
        in_specs=[
            pl.BlockSpec((tb, h), lambda i: (i, 0)),
            pl.BlockSpec((E, h), lambda i: (0, 0)),
        ],
        out_specs=[
            pl.BlockSpec((TOP_K, tb), lambda i: (0, i)),
            pl.BlockSpec((TOP_K, tb), lambda i: (0, i)),
        ],
        out_shape=[
            jax.ShapeDtypeStruct((TOP_K, t), jnp.int32),
            jax.ShapeDtypeStruct((TOP_K, t), jnp.float32),
        ],
    )(xs, W)
    return idx_t.T, wgt_t.T


# speedup vs baseline: 12.0695x; 1.1854x over previous
import functools

import jax
import jax.numpy as jnp
from jax.experimental import pallas as pl

E = 64
N_GROUP = 8
TOPK_GROUP = 3
TOP_K = 8
GROUP_SIZE = E // N_GROUP


def _gate_block(x_ref, w_ref, idx_ref, wgt_ref):
    x = x_ref[...]
    w = w_ref[...]
    lt = jax.lax.dot_general(
        w, x, (((1,), (1,)), ((), ())),
        preferred_element_type=jnp.float32)
    tb = lt.shape[1]
    ninf = jnp.float32(-jnp.inf)

    g = jnp.max(lt.reshape(N_GROUP, GROUP_SIZE, tb), axis=1)
    m0 = jnp.max(g, axis=0, keepdims=True)

    den = jnp.sum(jnp.exp(lt - m0), axis=0, keepdims=True)
    rden = 1.0 / den

    sel = jnp.zeros((N_GROUP, tb), dtype=jnp.float32)
    work = g
    for _ in range(TOPK_GROUP):
        gm = jnp.max(work, axis=0, keepdims=True)
        eq = work == gm
        sel = sel + jnp.where(eq, 1.0, 0.0)
        work = jnp.where(eq, ninf, work)

    sel64 = jnp.broadcast_to(
        sel.reshape(N_GROUP, 1, tb),
        (N_GROUP, GROUP_SIZE, tb)).reshape(E, tb)
    cand = jnp.where(sel64 > 0.0, lt, ninf)

    rowid = jax.lax.broadcasted_iota(jnp.int32, (E, tb), 0)
    work = cand
    for k in range(TOP_K):
        km = jnp.max(work, axis=0, keepdims=True)
        eq = work == km
        fidx = jnp.min(jnp.where(eq, rowid, E),
                       axis=0, keepdims=True)
        work = jnp.where(eq, ninf, work)
        idx_ref[k:k + 1, :] = fidx
        wgt_ref[k:k + 1, :] = jnp.exp(km - m0) * rden


@functools.partial(jax.jit, static_argnames=())
def kernel(x, W):
    b, s, h = x.shape
    t = b * s
    xs = x.reshape(t, h)
    tb = 2048
    grid = (t // tb,)
    idx_t, wgt_t = pl.pallas_call(
        _gate_block,
        grid=grid,
        in_specs=[
            pl.BlockSpec((tb, h), lambda i: (i, 0)),
            pl.BlockSpec((E, h), lambda i: (0, 0)),
        ],
        out_specs=[
            pl.BlockSpec((TOP_K, tb), lambda i: (0, i)),
            pl.BlockSpec((TOP_K, tb), lambda i: (0, i)),
        ],
        out_shape=[
            jax.ShapeDtypeStruct((TOP_K, t), jnp.int32),
            jax.ShapeDtypeStruct((TOP_K, t), jnp.float32),
        ],
    )(xs, W)
    return idx_t.T, wgt_t.T


# speedup vs baseline: 13.0157x; 1.0784x over previous
import functools

import jax
import jax.numpy as jnp
from jax.experimental import pallas as pl

E = 64
N_GROUP = 8
TOPK_GROUP = 3
TOP_K = 8
GROUP_SIZE = E // N_GROUP


def _gate_block(x_ref, w_ref, idx_ref, wgt_ref):
    x = x_ref[...]
    w = w_ref[...]
    lt = jax.lax.dot_general(
        w, x, (((1,), (1,)), ((), ())),
        preferred_element_type=jnp.float32)
    tb = lt.shape[1]
    ninf = jnp.float32(-jnp.inf)

    g = jnp.max(lt.reshape(N_GROUP, GROUP_SIZE, tb), axis=1)
    m0 = jnp.max(g, axis=0, keepdims=True)

    den = jnp.sum(jnp.exp(lt - m0), axis=0, keepdims=True)
    rden = 1.0 / den

    sel = jnp.zeros((N_GROUP, tb), dtype=jnp.float32)
    work = g
    for _ in range(TOPK_GROUP):
        gm = jnp.max(work, axis=0, keepdims=True)
        eq = work == gm
        sel = sel + jnp.where(eq, 1.0, 0.0)
        work = jnp.where(eq, ninf, work)

    sel64 = jnp.broadcast_to(
        sel.reshape(N_GROUP, 1, tb),
        (N_GROUP, GROUP_SIZE, tb)).reshape(E, tb)
    cand = jnp.where(sel64 > 0.0, lt, ninf)

    rowid = jax.lax.broadcasted_iota(jnp.int32, (E, tb), 0)
    work = cand
    for k in range(TOP_K):
        km = jnp.max(work, axis=0, keepdims=True)
        eq = work == km
        fidx = jnp.min(jnp.where(eq, rowid, E),
                       axis=0, keepdims=True)
        work = jnp.where(eq, ninf, work)
        idx_ref[k:k + 1, :] = fidx
        wgt_ref[k:k + 1, :] = jnp.exp(km - m0) * rden


@functools.partial(jax.jit, static_argnames=())
def kernel(x, W):
    b, s, h = x.shape
    t = b * s
    xs = x.reshape(t, h)
    tb = 4096
    grid = (t // tb,)
    idx_t, wgt_t = pl.pallas_call(
        _gate_block,
        grid=grid,
        in_specs=[
            pl.BlockSpec((tb, h), lambda i: (i, 0)),
            pl.BlockSpec((E, h), lambda i: (0, 0)),
        ],
        out_specs=[
            pl.BlockSpec((TOP_K, tb), lambda i: (0, i)),
            pl.BlockSpec((TOP_K, tb), lambda i: (0, i)),
        ],
        out_shape=[
            jax.ShapeDtypeStruct((TOP_K, t), jnp.int32),
            jax.ShapeDtypeStruct((TOP_K, t), jnp.float32),
        ],
    )(xs, W)
    return idx_t.T, wgt_t.T
